# Initial kernel scaffold; baseline (speedup 1.0000x reference)
import jax
import jax.numpy as jnp
from jax.experimental import pallas as pl


def kernel(source, target, encoder_outputs, encoder_hidden, embed, W_ih, W_hh, b_ih, W_out, b_out):
    raise NotImplementedError("write your pallas kernel here")


# speedup vs baseline: 4.6109x; 4.6109x over previous
import jax
import jax.numpy as jnp
from jax import lax
from jax.experimental import pallas as pl
from jax.experimental.pallas import tpu as pltpu

B = 32
K = 8
V = 1000
H = 1024
T = 16
SOS = 2
EOS = 3
BK = B * K
VP = 1024
NEG = -1e30
BIG = 1e9


def _decode_kernel(h0_ref, embed_ref, wih_ref, whh_ref, bih_ref, wout_ref,
                   bout_ref,
                   logsm_ref, sym_ref, pred_ref, tp0_ref, score_ref,
                   h_scr, iv_scr, seq_scr):
    f32 = jnp.float32
    i = pl.program_id(0)

    col = lax.broadcasted_iota(jnp.int32, (1, VP), 1)
    valid_col = col < V
    colf = col.astype(f32)
    rowf = lax.broadcasted_iota(jnp.int32, (1, BK), 1).astype(f32)
    beam3 = lax.broadcasted_iota(jnp.int32, (1, K, VP), 1).astype(f32)
    col3 = lax.broadcasted_iota(jnp.int32, (1, K, VP), 2)
    flatidx3 = jnp.where(col3 < V, beam3 * float(V) + col3.astype(f32), BIG)
    flatidx3 = jnp.broadcast_to(flatidx3, (B, K, VP))
    batch_base3 = lax.broadcasted_iota(jnp.int32, (B, K, 1), 0).astype(f32) * float(K)

    @pl.when(i == 0)
    def _init():
        h_scr[:] = h0_ref[:]
        iv_scr[:] = jnp.full((BK, 1), float(SOS), dtype=f32)
        r_id = lax.broadcasted_iota(jnp.int32, (BK, 1), 0)
        seq_scr[:] = jnp.where(r_id % K == 0, 0.0, -jnp.inf).astype(f32)

    h = h_scr[:]
    ivf = iv_scr[:]
    seq = seq_scr[:]

    onehot_x = (ivf == colf).astype(f32)
    x = jnp.dot(onehot_x, embed_ref[:], preferred_element_type=f32, precision=jax.lax.Precision.HIGHEST)

    r = jax.nn.sigmoid(jnp.dot(x, wih_ref[:, :H], preferred_element_type=f32)
                       + bih_ref[:, :H]
                       + jnp.dot(h, whh_ref[:, :H], preferred_element_type=f32))
    z = jax.nn.sigmoid(jnp.dot(x, wih_ref[:, H:2 * H], preferred_element_type=f32)
                       + bih_ref[:, H:2 * H]
                       + jnp.dot(h, whh_ref[:, H:2 * H], preferred_element_type=f32))
    n = jnp.tanh(jnp.dot(x, wih_ref[:, 2 * H:], preferred_element_type=f32)
                 + bih_ref[:, 2 * H:]
                 + r * jnp.dot(h, whh_ref[:, 2 * H:], preferred_element_type=f32))
    hnew = (1.0 - z) * n + z * h

    logits = jnp.dot(hnew, wout_ref[:], preferred_element_type=f32) + bout_ref[:]
    logits = jnp.where(valid_col, logits, NEG)
    m = jnp.max(logits, axis=1, keepdims=True)
    shifted = logits - m
    logsm = shifted - jnp.log(jnp.sum(jnp.exp(shifted), axis=1, keepdims=True))
    logsm_ref[:] = logsm.reshape(1, BK, VP)

    inflated = seq + logsm
    inflated = jnp.where(valid_col, inflated, -jnp.inf)
    sc3 = inflated.reshape(B, K, VP)

    chosen_list = []
    val_list = []
    for _ in range(K):
        m2 = jnp.max(sc3, axis=2, keepdims=True)
        m1 = jnp.max(m2, axis=1, keepdims=True)
        idxm = jnp.where(sc3 == m1, flatidx3, BIG)
        c2 = jnp.min(idxm, axis=2, keepdims=True)
        chosen = jnp.min(c2, axis=1, keepdims=True)
        chosen_list.append(chosen)
        val_list.append(m1)
        sc3 = jnp.where(flatidx3 == chosen, -jnp.inf, sc3)

    cand = jnp.concatenate(chosen_list, axis=1)
    vals = jnp.concatenate(val_list, axis=1)
    beam = jnp.floor(cand / float(V))
    vocab = cand - beam * float(V)
    pred = beam + batch_base3

    iv_new = vocab.reshape(BK, 1)
    pred_r = pred.reshape(BK, 1)
    scores_r = vals.reshape(BK, 1)

    sym_ref[:] = iv_new.reshape(1, BK, 1)
    pred_ref[:] = pred_r.reshape(1, BK, 1)

    oh_pred = (pred_r == rowf).astype(f32)
    h_scr[:] = jnp.dot(oh_pred, hnew, preferred_element_type=f32, precision=jax.lax.Precision.HIGHEST)
    iv_scr[:] = iv_new
    seq_scr[:] = jnp.where(iv_new == float(EOS), -jnp.inf, scores_r)

    @pl.when(i == T - 1)
    def _final_sort():
        fs = scores_r.reshape(B, K, 1)
        kidx = lax.broadcasted_iota(jnp.int32, (B, K, 1), 1).astype(f32)
        w = fs
        sidx_list = []
        sval_list = []
        for _ in range(K):
            m1 = jnp.max(w, axis=1, keepdims=True)
            sel = jnp.min(jnp.where(w == m1, kidx, BIG), axis=1, keepdims=True)
            sidx_list.append(sel)
            sval_list.append(m1)
            w = jnp.where(kidx == sel, -jnp.inf, w)
        sval = jnp.concatenate(sval_list, axis=1)
        sidx = jnp.concatenate(sidx_list, axis=1)
        pad = jnp.zeros((B, 128 - K, 1), dtype=f32)
        score_ref[:] = jnp.concatenate([sval, pad], axis=1).reshape(B, 128)
        tp0_ref[:] = (sidx + batch_base3).reshape(BK, 1)


def _backtrack_kernel(logsm_ref, sym_ref, pred_ref, tp0_ref,
                      outs_ref, seqs_ref,
                      tp_scr):
    f32 = jnp.float32
    i = pl.program_id(0)
    rowf = lax.broadcasted_iota(jnp.int32, (1, BK), 1).astype(f32)

    @pl.when(i == 0)
    def _init():
        tp_scr[:] = tp0_ref[:]

    tp = tp_scr[:]
    ohp = (tp == rowf).astype(f32)
    cur_sym = jnp.dot(ohp, sym_ref[0], preferred_element_type=f32, precision=jax.lax.Precision.HIGHEST)
    new_tp = jnp.dot(ohp, pred_ref[0], preferred_element_type=f32, precision=jax.lax.Precision.HIGHEST)
    seqs_ref[:] = cur_sym.astype(jnp.int32).reshape(1, BK, 1)

    tp32 = tp.reshape(B, K, 1)[:, 0, :]
    oh32 = (tp32 == rowf).astype(f32)
    outs_ref[:] = jnp.dot(oh32, logsm_ref[0],
                          preferred_element_type=f32, precision=jax.lax.Precision.HIGHEST).reshape(1, B, VP)
    tp_scr[:] = new_tp


@jax.jit
def _run(h0, embed_p, W_ih, W_hh, b_ih, W_out_p, b_out_p):
    logsm_all, sym_all, pred_all, tp0, score = pl.pallas_call(
        _decode_kernel,
        grid=(T,),
        in_specs=[
            pl.BlockSpec((BK, H), lambda i: (0, 0)),
            pl.BlockSpec((VP, H), lambda i: (0, 0)),
            pl.BlockSpec((H, 3 * H), lambda i: (0, 0)),
            pl.BlockSpec((H, 3 * H), lambda i: (0, 0)),
            pl.BlockSpec((1, 3 * H), lambda i: (0, 0)),
            pl.BlockSpec((H, VP), lambda i: (0, 0)),
            pl.BlockSpec((1, VP), lambda i: (0, 0)),
        ],
        out_specs=[
            pl.BlockSpec((1, BK, VP), lambda i: (i, 0, 0)),
            pl.BlockSpec((1, BK, 1), lambda i: (i, 0, 0)),
            pl.BlockSpec((1, BK, 1), lambda i: (i, 0, 0)),
            pl.BlockSpec((BK, 1), lambda i: (0, 0)),
            pl.BlockSpec((B, 128), lambda i: (0, 0)),
        ],
        out_shape=[
            jax.ShapeDtypeStruct((T, BK, VP), jnp.float32),
            jax.ShapeDtypeStruct((T, BK, 1), jnp.float32),
            jax.ShapeDtypeStruct((T, BK, 1), jnp.float32),
            jax.ShapeDtypeStruct((BK, 1), jnp.float32),
            jax.ShapeDtypeStruct((B, 128), jnp.float32),
        ],
        scratch_shapes=[
            pltpu.VMEM((BK, H), jnp.float32),
            pltpu.VMEM((BK, 1), jnp.float32),
            pltpu.VMEM((BK, 1), jnp.float32),
        ],
    )(h0, embed_p, W_ih, W_hh, b_ih, W_out_p, b_out_p)

    outs, seqs = pl.pallas_call(
        _backtrack_kernel,
        grid=(T,),
        in_specs=[
            pl.BlockSpec((1, BK, VP), lambda i: (T - 1 - i, 0, 0)),
            pl.BlockSpec((1, BK, 1), lambda i: (T - 1 - i, 0, 0)),
            pl.BlockSpec((1, BK, 1), lambda i: (T - 1 - i, 0, 0)),
            pl.BlockSpec((BK, 1), lambda i: (0, 0)),
        ],
        out_specs=[
            pl.BlockSpec((1, B, VP), lambda i: (T - 1 - i, 0, 0)),
            pl.BlockSpec((1, BK, 1), lambda i: (T - 1 - i, 0, 0)),
        ],
        out_shape=[
            jax.ShapeDtypeStruct((T, B, VP), jnp.float32),
            jax.ShapeDtypeStruct((T, BK, 1), jnp.int32),
        ],
        scratch_shapes=[
            pltpu.VMEM((BK, 1), jnp.float32),
        ],
    )(logsm_all, sym_all, pred_all, tp0)
    return outs, seqs, score


def kernel(source, target, encoder_outputs, encoder_hidden, embed, W_ih, W_hh,
           b_ih, W_out, b_out):
    del source, target, encoder_outputs
    h0 = jnp.tile(encoder_hidden, (1, K, 1))[0]
    embed_p = jnp.pad(embed, ((0, VP - V), (0, 0)))
    W_out_p = jnp.pad(W_out, ((0, 0), (0, VP - V)))
    b_out_p = jnp.pad(b_out, ((0, VP - V),)).reshape(1, VP)
    b_ih_r = b_ih.reshape(1, 3 * H)
    outs, seqs, score = _run(h0, embed_p, W_ih, W_hh, b_ih_r, W_out_p, b_out_p)
    return (outs[:, :, :V],
            seqs.reshape(T, B, K),
            score[:, :K])


# speedup vs baseline: 5.0788x; 1.1015x over previous
import jax
import jax.numpy as jnp
from jax import lax
from jax.experimental import pallas as pl
from jax.experimental.pallas import tpu as pltpu

B = 32
K = 8
V = 1000
H = 1024
T = 16
SOS = 2
EOS = 3
BK = B * K
VP = 1024
NEG = -1e30
BIG = 1e9


def _decode_kernel(h0_ref, embed_ref, wih_ref, whh_ref, bih_ref, wout_ref,
                   bout_ref,
                   logsm_ref, sym_ref, pred_ref, tp0_ref, score_ref,
                   h_scr, iv_scr, seq_scr):
    f32 = jnp.float32
    i = pl.program_id(0)

    col = lax.broadcasted_iota(jnp.int32, (1, VP), 1)
    valid_col = col < V
    colf = col.astype(f32)
    rowf = lax.broadcasted_iota(jnp.int32, (1, BK), 1).astype(f32)
    beam3 = lax.broadcasted_iota(jnp.int32, (1, K, VP), 1).astype(f32)
    col3 = lax.broadcasted_iota(jnp.int32, (1, K, VP), 2)
    flatidx3 = jnp.where(col3 < V, beam3 * float(V) + col3.astype(f32), BIG)
    flatidx3 = jnp.broadcast_to(flatidx3, (B, K, VP))
    batch_base3 = lax.broadcasted_iota(jnp.int32, (B, K, 1), 0).astype(f32) * float(K)

    @pl.when(i == 0)
    def _init():
        h_scr[:] = h0_ref[:]
        iv_scr[:] = jnp.full((BK, 1), float(SOS), dtype=f32)
        r_id = lax.broadcasted_iota(jnp.int32, (BK, 1), 0)
        seq_scr[:] = jnp.where(r_id % K == 0, 0.0, -jnp.inf).astype(f32)

    h = h_scr[:]
    ivf = iv_scr[:]
    seq = seq_scr[:]

    onehot_x = (ivf == colf).astype(f32)
    onehot3 = jnp.concatenate([onehot_x, onehot_x, onehot_x], axis=1)
    x = jnp.dot(onehot3, embed_ref[:], preferred_element_type=f32)

    r = jax.nn.sigmoid(jnp.dot(x, wih_ref[:, :H], preferred_element_type=f32)
                       + bih_ref[:, :H]
                       + jnp.dot(h, whh_ref[:, :H], preferred_element_type=f32))
    z = jax.nn.sigmoid(jnp.dot(x, wih_ref[:, H:2 * H], preferred_element_type=f32)
                       + bih_ref[:, H:2 * H]
                       + jnp.dot(h, whh_ref[:, H:2 * H], preferred_element_type=f32))
    n = jnp.tanh(jnp.dot(x, wih_ref[:, 2 * H:], preferred_element_type=f32)
                 + bih_ref[:, 2 * H:]
                 + r * jnp.dot(h, whh_ref[:, 2 * H:], preferred_element_type=f32))
    hnew = (1.0 - z) * n + z * h

    logits = jnp.dot(hnew, wout_ref[:], preferred_element_type=f32) + bout_ref[:]
    logits = jnp.where(valid_col, logits, NEG)
    m = jnp.max(logits, axis=1, keepdims=True)
    shifted = logits - m
    logsm = shifted - jnp.log(jnp.sum(jnp.exp(shifted), axis=1, keepdims=True))
    logsm_ref[:] = logsm.reshape(1, BK, VP)

    inflated = seq + logsm
    inflated = jnp.where(valid_col, inflated, -jnp.inf)
    sc3 = inflated.reshape(B, K, VP)

    chosen_list = []
    val_list = []
    for _ in range(K):
        m2 = jnp.max(sc3, axis=2, keepdims=True)
        m1 = jnp.max(m2, axis=1, keepdims=True)
        idxm = jnp.where(sc3 == m1, flatidx3, BIG)
        c2 = jnp.min(idxm, axis=2, keepdims=True)
        chosen = jnp.min(c2, axis=1, keepdims=True)
        chosen_list.append(chosen)
        val_list.append(m1)
        sc3 = jnp.where(flatidx3 == chosen, -jnp.inf, sc3)

    cand = jnp.concatenate(chosen_list, axis=1)
    vals = jnp.concatenate(val_list, axis=1)
    beam = jnp.floor(cand / float(V))
    vocab = cand - beam * float(V)
    pred = beam + batch_base3

    iv_new = vocab.reshape(BK, 1)
    pred_r = pred.reshape(BK, 1)
    scores_r = vals.reshape(BK, 1)

    sym_ref[:] = iv_new.reshape(1, BK, 1)
    pred_ref[:] = pred_r.reshape(1, BK, 1)

    oh_pred = (pred_r == rowf).astype(f32)
    h_scr[:] = jnp.dot(oh_pred, hnew, preferred_element_type=f32, precision=jax.lax.Precision.HIGHEST)
    iv_scr[:] = iv_new
    seq_scr[:] = jnp.where(iv_new == float(EOS), -jnp.inf, scores_r)

    @pl.when(i == T - 1)
    def _final_sort():
        fs = scores_r.reshape(B, K, 1)
        kidx = lax.broadcasted_iota(jnp.int32, (B, K, 1), 1).astype(f32)
        w = fs
        sidx_list = []
        sval_list = []
        for _ in range(K):
            m1 = jnp.max(w, axis=1, keepdims=True)
            sel = jnp.min(jnp.where(w == m1, kidx, BIG), axis=1, keepdims=True)
            sidx_list.append(sel)
            sval_list.append(m1)
            w = jnp.where(kidx == sel, -jnp.inf, w)
        sval = jnp.concatenate(sval_list, axis=1)
        sidx = jnp.concatenate(sidx_list, axis=1)
        pad = jnp.zeros((B, 128 - K, 1), dtype=f32)
        score_ref[:] = jnp.concatenate([sval, pad], axis=1).reshape(B, 128)
        tp0_ref[:] = (sidx + batch_base3).reshape(BK, 1)


def _backtrack_kernel(logsm_ref, sym_ref, pred_ref, tp0_ref,
                      outs_ref, seqs_ref,
                      tp_scr):
    f32 = jnp.float32
    i = pl.program_id(0)
    rowf = lax.broadcasted_iota(jnp.int32, (1, BK), 1).astype(f32)

    @pl.when(i == 0)
    def _init():
        tp_scr[:] = tp0_ref[:]

    tp = tp_scr[:]
    ohp = (tp == rowf).astype(f32)
    cur_sym = jnp.dot(ohp, sym_ref[0], preferred_element_type=f32, precision=jax.lax.Precision.HIGHEST)
    new_tp = jnp.dot(ohp, pred_ref[0], preferred_element_type=f32, precision=jax.lax.Precision.HIGHEST)
    seqs_ref[:] = cur_sym.astype(jnp.int32).reshape(1, BK, 1)

    tp32 = tp.reshape(B, K, 1)[:, 0, :]
    oh32 = (tp32 == rowf).astype(f32)
    outs_ref[:] = jnp.dot(oh32, logsm_ref[0],
                          preferred_element_type=f32, precision=jax.lax.Precision.HIGHEST).reshape(1, B, VP)
    tp_scr[:] = new_tp


@jax.jit
def _run(h0, embed_p, W_ih, W_hh, b_ih, W_out_p, b_out_p):
    logsm_all, sym_all, pred_all, tp0, score = pl.pallas_call(
        _decode_kernel,
        grid=(T,),
        in_specs=[
            pl.BlockSpec((BK, H), lambda i: (0, 0)),
            pl.BlockSpec((3 * VP, H), lambda i: (0, 0)),
            pl.BlockSpec((H, 3 * H), lambda i: (0, 0)),
            pl.BlockSpec((H, 3 * H), lambda i: (0, 0)),
            pl.BlockSpec((1, 3 * H), lambda i: (0, 0)),
            pl.BlockSpec((H, VP), lambda i: (0, 0)),
            pl.BlockSpec((1, VP), lambda i: (0, 0)),
        ],
        out_specs=[
            pl.BlockSpec((1, BK, VP), lambda i: (i, 0, 0)),
            pl.BlockSpec((1, BK, 1), lambda i: (i, 0, 0)),
            pl.BlockSpec((1, BK, 1), lambda i: (i, 0, 0)),
            pl.BlockSpec((BK, 1), lambda i: (0, 0)),
            pl.BlockSpec((B, 128), lambda i: (0, 0)),
        ],
        out_shape=[
            jax.ShapeDtypeStruct((T, BK, VP), jnp.float32),
            jax.ShapeDtypeStruct((T, BK, 1), jnp.float32),
            jax.ShapeDtypeStruct((T, BK, 1), jnp.float32),
            jax.ShapeDtypeStruct((BK, 1), jnp.float32),
            jax.ShapeDtypeStruct((B, 128), jnp.float32),
        ],
        scratch_shapes=[
            pltpu.VMEM((BK, H), jnp.float32),
            pltpu.VMEM((BK, 1), jnp.float32),
            pltpu.VMEM((BK, 1), jnp.float32),
        ],
    )(h0, embed_p, W_ih, W_hh, b_ih, W_out_p, b_out_p)

    outs, seqs = pl.pallas_call(
        _backtrack_kernel,
        grid=(T,),
        in_specs=[
            pl.BlockSpec((1, BK, VP), lambda i: (T - 1 - i, 0, 0)),
            pl.BlockSpec((1, BK, 1), lambda i: (T - 1 - i, 0, 0)),
            pl.BlockSpec((1, BK, 1), lambda i: (T - 1 - i, 0, 0)),
            pl.BlockSpec((BK, 1), lambda i: (0, 0)),
        ],
        out_specs=[
            pl.BlockSpec((1, B, VP), lambda i: (T - 1 - i, 0, 0)),
            pl.BlockSpec((1, BK, 1), lambda i: (T - 1 - i, 0, 0)),
        ],
        out_shape=[
            jax.ShapeDtypeStruct((T, B, VP), jnp.float32),
            jax.ShapeDtypeStruct((T, BK, 1), jnp.int32),
        ],
        scratch_shapes=[
            pltpu.VMEM((BK, 1), jnp.float32),
        ],
    )(logsm_all, sym_all, pred_all, tp0)
    return outs, seqs, score


def kernel(source, target, encoder_outputs, encoder_hidden, embed, W_ih, W_hh,
           b_ih, W_out, b_out):
    del source, target, encoder_outputs
    h0 = jnp.tile(encoder_hidden, (1, K, 1))[0]
    e = jnp.pad(embed, ((0, VP - V), (0, 0)))
    e_hi = e.astype(jnp.bfloat16).astype(jnp.float32)
    e_mid = (e - e_hi).astype(jnp.bfloat16).astype(jnp.float32)
    e_lo = e - e_hi - e_mid
    embed_p = jnp.concatenate([e_hi, e_mid, e_lo], axis=0)
    W_out_p = jnp.pad(W_out, ((0, 0), (0, VP - V)))
    b_out_p = jnp.pad(b_out, ((0, VP - V),)).reshape(1, VP)
    b_ih_r = b_ih.reshape(1, 3 * H)
    outs, seqs, score = _run(h0, embed_p, W_ih, W_hh, b_ih_r, W_out_p, b_out_p)
    return (outs[:, :, :V],
            seqs.reshape(T, B, K),
            score[:, :K])
